# baseline (speedup 1.0000x reference)
import jax
import jax.numpy as jnp
from jax import lax
from jax.experimental import pallas as pl
from jax.experimental.pallas import tpu as pltpu
from jax.experimental.pallas import tpu_sc as plsc

VOCAB = 1000000
D = 64
DP = 128
B = 16384
C = 6
L = 16
NC, NS = 2, 16
NW = NC * NS
BW = B // NW
CB = 64
NCH = BW // CB
FC = CB * C


def _make_kernel():
    mesh = plsc.VectorSubcoreMesh(
        core_axis_name="c", subcore_axis_name="s",
        num_cores=NC, num_subcores=NS)

    def body(tgt_hbm, ctx_hbm, ttab_hbm, ctab_hbm, out_hbm,
             tidx_v, cidx_v, tidxh_v, cidxh_v,
             trows0, trows1, crows0, crows1, outv0, outv1,
             gsem0, gsem1, osem0, osem1):
        cid = lax.axis_index("c")
        sid = lax.axis_index("s")
        wid = sid * NC + cid
        base = wid * BW

        pltpu.sync_copy(tgt_hbm.at[pl.ds(base, BW)], tidx_v)
        pltpu.sync_copy(ctx_hbm.at[pl.ds(base * C, BW * C)], cidx_v)

        def halve(src, dst, n):
            def hb(i, c):
                dst[pl.ds(i * L, L)] = src[pl.ds(i * L, L)] >> 1
                return c
            lax.fori_loop(0, n // L, hb, 0)
        halve(tidx_v, tidxh_v, BW)
        halve(cidx_v, cidxh_v, BW * C)

        trows = (trows0, trows1)
        crows = (crows0, crows1)
        outv = (outv0, outv1)
        gsem = (gsem0, gsem1)
        osem = (osem0, osem1)

        out_handles = [None] * NCH

        def issue(k):
            s = k % 2
            pltpu.async_copy(
                ttab_hbm.at[tidxh_v.at[pl.ds(k * CB, CB)]], trows[s], gsem[s])
            for j in range(FC // 128):
                pltpu.async_copy(
                    ctab_hbm.at[cidxh_v.at[pl.ds(k * FC + j * 128, 128)]],
                    crows[s].at[pl.ds(j * 128, 128)], gsem[s])

        def wait_rows(s):
            pltpu.make_async_copy(
                ttab_hbm.at[pl.ds(0, CB)], trows[s], gsem[s]).wait()
            pltpu.make_async_copy(
                ctab_hbm.at[pl.ds(0, FC)], crows[s], gsem[s]).wait()

        lanes = lax.iota(jnp.int32, L)
        one = jnp.ones((L,), jnp.int32)

        def compute(k, tr, cr, ov):
            def gbody(g, carry):
                f0 = g * L
                floc = f0 + lanes
                bloc = floc // C
                tpar = plsc.load_gather(tidx_v, [k * CB + bloc]) & one
                cpar = plsc.load_gather(cidx_v, [k * FC + floc]) & one
                wcol = tpar * D
                ccol = cpar * D
                acc = jnp.zeros((L,), jnp.float32)
                for d in range(D):
                    wv = plsc.load_gather(tr, [bloc, wcol + d])
                    cv = plsc.load_gather(cr, [floc, ccol + d])
                    acc += wv * cv
                ov[pl.ds(f0, L)] = acc
                return carry
            lax.fori_loop(0, FC // L, gbody, 0)

        issue(0)
        for k in range(NCH):
            s = k % 2
            if k + 1 < NCH:
                issue(k + 1)
            wait_rows(s)
            if k >= 2:
                out_handles[k - 2].wait()
            compute(k, trows[s], crows[s], outv[s])
            out_handles[k] = pltpu.async_copy(
                outv[s], out_hbm.at[pl.ds((base + k * CB) * C, FC)],
                osem[s])
        out_handles[NCH - 2].wait()
        out_handles[NCH - 1].wait()

    return pl.kernel(
        body,
        out_type=jax.ShapeDtypeStruct((B * C,), jnp.float32),
        mesh=mesh,
        compiler_params=pltpu.CompilerParams(needs_layout_passes=False),
        scratch_types=[
            pltpu.VMEM((BW,), jnp.int32),
            pltpu.VMEM((BW * C,), jnp.int32),
            pltpu.VMEM((BW,), jnp.int32),
            pltpu.VMEM((BW * C,), jnp.int32),
            pltpu.VMEM((CB, DP), jnp.float32),
            pltpu.VMEM((CB, DP), jnp.float32),
            pltpu.VMEM((FC, DP), jnp.float32),
            pltpu.VMEM((FC, DP), jnp.float32),
            pltpu.VMEM((FC,), jnp.float32),
            pltpu.VMEM((FC,), jnp.float32),
            pltpu.SemaphoreType.DMA,
            pltpu.SemaphoreType.DMA,
            pltpu.SemaphoreType.DMA,
            pltpu.SemaphoreType.DMA,
        ],
    )


_w2v = _make_kernel()


def kernel(target, context, target_table, context_table):
    if target.ndim == 2:
        target = jnp.squeeze(target, axis=1)
    tgt = target.astype(jnp.int32)
    ctx = context.astype(jnp.int32).reshape(B * C)
    tt = target_table.reshape(VOCAB // 2, 2 * D)
    ct = context_table.reshape(VOCAB // 2, 2 * D)
    return _w2v(tgt, ctx, tt, ct).reshape(B, C)


# speedup vs baseline: 1.7460x; 1.7460x over previous
import jax
import jax.numpy as jnp
from jax import lax
from jax.experimental import pallas as pl
from jax.experimental.pallas import tpu as pltpu
from jax.experimental.pallas import tpu_sc as plsc

VOCAB = 1000000
D = 64
B = 16384
C = 6
L = 16
NC, NS = 2, 16
NW = NC * NS
BW = B // NW
CB = 64
NCH = BW // CB
FC = CB * C


def _make_kernel():
    mesh = plsc.VectorSubcoreMesh(
        core_axis_name="c", subcore_axis_name="s",
        num_cores=NC, num_subcores=NS)

    def body(tgt_hbm, ctx_hbm, ttab_hbm, ctab_hbm, out_hbm,
             tidx_v, cidx_v,
             trows0, trows1, crows0, crows1, outv0, outv1,
             gsem0, gsem1, osem0, osem1):
        cid = lax.axis_index("c")
        sid = lax.axis_index("s")
        wid = sid * NC + cid
        base = wid * BW

        pltpu.sync_copy(tgt_hbm.at[pl.ds(base, BW)], tidx_v)
        pltpu.sync_copy(ctx_hbm.at[pl.ds(base * C, BW * C)], cidx_v)

        trows = (trows0, trows1)
        crows = (crows0, crows1)
        outv = (outv0, outv1)
        gsem = (gsem0, gsem1)
        osem = (osem0, osem1)

        out_handles = [None] * NCH

        def issue(k):
            s = k % 2

            def tg(g, c):
                iv = tidx_v[pl.ds(k * CB + g * L, L)]
                for j in range(L):
                    pltpu.async_copy(
                        ttab_hbm.at[pl.ds(iv[j], 1), :],
                        trows[s].at[pl.ds(g * L + j, 1), :], gsem[s])
                return c
            lax.fori_loop(0, CB // L, tg, 0)

            def cg(g, c):
                iv = cidx_v[pl.ds(k * FC + g * L, L)]
                for j in range(L):
                    pltpu.async_copy(
                        ctab_hbm.at[pl.ds(iv[j], 1), :],
                        crows[s].at[pl.ds(g * L + j, 1), :], gsem[s])
                return c
            lax.fori_loop(0, FC // L, cg, 0)

        def wait_rows(s):
            pltpu.make_async_copy(
                ttab_hbm.at[pl.ds(0, CB), :], trows[s], gsem[s]).wait()
            pltpu.make_async_copy(
                ctab_hbm.at[pl.ds(0, FC), :], crows[s], gsem[s]).wait()

        lanes = lax.iota(jnp.int32, L)
        mask6 = lanes < C
        cmasks = [lanes == c for c in range(C)]
        perms = [(lanes ^ k)[:, None] for k in (8, 4, 2, 1)]
        dnums = lax.GatherDimensionNumbers(
            offset_dims=(), collapsed_slice_dims=(0,), start_index_map=(0,))

        def lanesum(p):
            for perm in perms:
                g = lax.gather(p, perm, dnums, slice_sizes=(1,),
                               mode=lax.GatherScatterMode.PROMISE_IN_BOUNDS)
                p = p + g
            return p

        def compute(tr, cr, ov):
            def bbody(b, carry):
                w0 = tr[b, pl.ds(0, L)]
                w1 = tr[b, pl.ds(L, L)]
                w2 = tr[b, pl.ds(2 * L, L)]
                w3 = tr[b, pl.ds(3 * L, L)]
                r0 = b * C
                acc = jnp.zeros((L,), jnp.float32)
                for c in range(C):
                    r = r0 + c
                    p = w0 * cr[r, pl.ds(0, L)]
                    p += w1 * cr[r, pl.ds(L, L)]
                    p += w2 * cr[r, pl.ds(2 * L, L)]
                    p += w3 * cr[r, pl.ds(3 * L, L)]
                    acc = jnp.where(cmasks[c], lanesum(p), acc)
                plsc.store_scatter(ov, [r0 + lanes], acc, mask=mask6)
                return carry
            lax.fori_loop(0, CB, bbody, 0)

        issue(0)
        for k in range(NCH):
            s = k % 2
            if k + 1 < NCH:
                issue(k + 1)
            wait_rows(s)
            if k >= 2:
                out_handles[k - 2].wait()
            compute(trows[s], crows[s], outv[s])
            out_handles[k] = pltpu.async_copy(
                outv[s], out_hbm.at[pl.ds((base + k * CB) * C, FC)],
                osem[s])
        out_handles[NCH - 2].wait()
        out_handles[NCH - 1].wait()

    return pl.kernel(
        body,
        out_type=jax.ShapeDtypeStruct((B * C,), jnp.float32),
        mesh=mesh,
        compiler_params=pltpu.CompilerParams(needs_layout_passes=False),
        scratch_types=[
            pltpu.VMEM((BW,), jnp.int32),
            pltpu.VMEM((BW * C,), jnp.int32),
            pltpu.VMEM((CB, D), jnp.float32),
            pltpu.VMEM((CB, D), jnp.float32),
            pltpu.VMEM((CB * C, D), jnp.float32),
            pltpu.VMEM((CB * C, D), jnp.float32),
            pltpu.VMEM((FC,), jnp.float32),
            pltpu.VMEM((FC,), jnp.float32),
            pltpu.SemaphoreType.DMA,
            pltpu.SemaphoreType.DMA,
            pltpu.SemaphoreType.DMA,
            pltpu.SemaphoreType.DMA,
        ],
    )


_w2v = _make_kernel()


def kernel(target, context, target_table, context_table):
    if target.ndim == 2:
        target = jnp.squeeze(target, axis=1)
    tgt = target.astype(jnp.int32)
    ctx = context.astype(jnp.int32).reshape(B * C)
    return _w2v(tgt, ctx, target_table, context_table).reshape(B, C)
